# Initial kernel scaffold; baseline (speedup 1.0000x reference)
import jax
import jax.numpy as jnp
from jax.experimental import pallas as pl


def kernel(x, edge_index, W1, b1, W2, b2):
    raise NotImplementedError("write your pallas kernel here")


# speedup vs baseline: 31.8969x; 31.8969x over previous
import functools

import jax
import jax.numpy as jnp
from jax import lax
from jax.experimental import pallas as pl
from jax.experimental.pallas import tpu as pltpu
from jax.experimental.pallas import tpu_sc as plsc

HIDDEN = 16
N_OUT = 2

NC = 2
NS = 16
NW = NC * NS
WIN = 128

_MESH = plsc.VectorSubcoreMesh(core_axis_name="c", subcore_axis_name="s")


def _sc_degree(dst2d, wpw, npad):
    per_tile = npad // NS

    @functools.partial(
        pl.kernel,
        out_type=jax.ShapeDtypeStruct((NC, npad), jnp.float32),
        mesh=_MESH,
        compiler_params=pltpu.CompilerParams(use_tc_tiling_on_sc=False),
        scratch_types=[
            pltpu.VMEM_SHARED((npad,), jnp.float32),
            pltpu.VMEM((wpw, WIN), jnp.int32),
            pltpu.VMEM((WIN,), jnp.float32),
            pltpu.VMEM((per_tile,), jnp.float32),
        ],
    )
    def deg_kernel(dst_hbm, out_hbm, accum, idx_v, ones_v, zb):
        cid = lax.axis_index("c")
        sid = lax.axis_index("s")
        w = cid * NS + sid

        @pl.loop(0, per_tile // 16)
        def _(i):
            zb[pl.ds(i * 16, 16)] = jnp.zeros((16,), jnp.float32)

        @pl.loop(0, WIN // 16)
        def _(i):
            ones_v[pl.ds(i * 16, 16)] = jnp.full((16,), 1.0, jnp.float32)

        pltpu.sync_copy(zb, accum.at[pl.ds(sid * per_tile, per_tile)])
        pltpu.sync_copy(dst_hbm.at[pl.ds(w * wpw, wpw)], idx_v)
        plsc.subcore_barrier()

        @pl.loop(0, wpw)
        def _(j):
            pltpu.sync_copy(ones_v, accum.at[idx_v.at[j]], add=True)

        plsc.subcore_barrier()
        pltpu.sync_copy(accum.at[pl.ds(sid * per_tile, per_tile)],
                        out_hbm.at[cid, pl.ds(sid * per_tile, per_tile)])

    return deg_kernel(dst2d)


def _sc_aggregate(table, src2d, dst2d, wpw, npad):
    per_tile = npad // NS

    @functools.partial(
        pl.kernel,
        out_type=jax.ShapeDtypeStruct((NC, npad, HIDDEN), jnp.float32),
        mesh=_MESH,
        compiler_params=pltpu.CompilerParams(use_tc_tiling_on_sc=False),
        scratch_types=[
            pltpu.VMEM_SHARED((npad, HIDDEN), jnp.float32),
            pltpu.VMEM((wpw, WIN), jnp.int32),
            pltpu.VMEM((wpw, WIN), jnp.int32),
            pltpu.VMEM((WIN, HIDDEN), jnp.float32),
            pltpu.VMEM((per_tile, HIDDEN), jnp.float32),
        ],
    )
    def agg_kernel(tab_hbm, src_hbm, dst_hbm, out_hbm,
                   accum, src_v, dst_v, rows_v, zb):
        cid = lax.axis_index("c")
        sid = lax.axis_index("s")
        w = cid * NS + sid

        @pl.loop(0, per_tile)
        def _(i):
            zb[i, :] = jnp.zeros((HIDDEN,), jnp.float32)

        pltpu.sync_copy(zb, accum.at[pl.ds(sid * per_tile, per_tile)])
        pltpu.sync_copy(src_hbm.at[pl.ds(w * wpw, wpw)], src_v)
        pltpu.sync_copy(dst_hbm.at[pl.ds(w * wpw, wpw)], dst_v)
        plsc.subcore_barrier()

        @pl.loop(0, wpw)
        def _(j):
            pltpu.sync_copy(tab_hbm.at[src_v.at[j]], rows_v)
            pltpu.sync_copy(rows_v, accum.at[dst_v.at[j]], add=True)

        plsc.subcore_barrier()
        pltpu.sync_copy(accum.at[pl.ds(sid * per_tile, per_tile)],
                        out_hbm.at[cid, pl.ds(sid * per_tile, per_tile)])

    return agg_kernel(table, src2d, dst2d)


def _tc_matmul(xp, W1):
    npad = xp.shape[0]

    def body(x_ref, w_ref, o_ref):
        o_ref[...] = lax.dot_general(
            x_ref[...], w_ref[...], (((1,), (1,)), ((), ())),
            preferred_element_type=jnp.float32)

    return pl.pallas_call(
        body,
        out_shape=jax.ShapeDtypeStruct((npad, HIDDEN), jnp.float32),
    )(xp, W1)


def _tc_scale(degT, h1):
    npad = h1.shape[0]

    def body(d_ref, h_ref, g_ref, dis_ref):
        deg = d_ref[:, 0:1] + d_ref[:, 1:2] + 1.0
        dis = lax.rsqrt(deg)
        dis_ref[...] = dis
        g_ref[...] = dis * h_ref[...]

    return pl.pallas_call(
        body,
        out_shape=(jax.ShapeDtypeStruct((npad, HIDDEN), jnp.float32),
                   jax.ShapeDtypeStruct((npad, 1), jnp.float32)),
    )(degT, h1)


def _tc_mid(P, g1, dis, b1row):
    npad = g1.shape[0]

    def body(p_ref, g_ref, dis_ref, b_ref, o_ref):
        agg = p_ref[0] + p_ref[1] + g_ref[...]
        pre = dis_ref[...] * agg + b_ref[...]
        z = jnp.where(pre >= 0, pre, 0.01 * pre)
        o_ref[...] = dis_ref[...] * z

    return pl.pallas_call(
        body,
        out_shape=jax.ShapeDtypeStruct((npad, HIDDEN), jnp.float32),
    )(P, g1, dis, b1row)


def _tc_final(Q, g2, dis, W2, b2row):
    npad = g2.shape[0]

    def body(q_ref, g_ref, dis_ref, w_ref, b_ref, o_ref):
        agg = dis_ref[...] * (q_ref[0] + q_ref[1] + g_ref[...])
        logits = lax.dot_general(
            agg, w_ref[...], (((1,), (1,)), ((), ())),
            preferred_element_type=jnp.float32) + b_ref[...]
        m = jnp.max(logits, axis=1, keepdims=True)
        s = jnp.sum(jnp.exp(logits - m), axis=1, keepdims=True)
        o_ref[...] = logits - (m + jnp.log(s))

    return pl.pallas_call(
        body,
        out_shape=jax.ShapeDtypeStruct((npad, N_OUT), jnp.float32),
    )(Q, g2, dis, W2, b2row)


def kernel(x, edge_index, W1, b1, W2, b2):
    n = x.shape[0]
    e = edge_index.shape[1]
    npad = -(-n // (NS * 16)) * (NS * 16)
    wpw = -(-e // (NW * WIN))
    wpw = -(-wpw // 8) * 8
    ep = wpw * NW * WIN

    ei = edge_index.astype(jnp.int32)
    pad = jnp.full((ep - e,), n, jnp.int32)
    src2d = jnp.concatenate([ei[0], pad]).reshape(NW * wpw, WIN)
    dst2d = jnp.concatenate([ei[1], pad]).reshape(NW * wpw, WIN)
    xp = jnp.pad(x, ((0, npad - n), (0, 0)))

    degh = _sc_degree(dst2d, wpw, npad)
    h1 = _tc_matmul(xp, W1)
    g1, dis = _tc_scale(degh.T, h1)
    P = _sc_aggregate(g1, src2d, dst2d, wpw, npad)
    g2 = _tc_mid(P, g1, dis, b1.reshape(1, HIDDEN))
    Q = _sc_aggregate(g2, src2d, dst2d, wpw, npad)
    out = _tc_final(Q, g2, dis, W2, b2.reshape(1, N_OUT))
    return out[:n]


# speedup vs baseline: 40.7457x; 1.2774x over previous
import functools

import jax
import jax.numpy as jnp
from jax import lax
from jax.experimental import pallas as pl
from jax.experimental.pallas import tpu as pltpu
from jax.experimental.pallas import tpu_sc as plsc

HIDDEN = 16
N_OUT = 2

NC = 2
NS = 16
NW = NC * NS
WIN = 128
GRP = 4

_MESH = plsc.VectorSubcoreMesh(core_axis_name="c", subcore_axis_name="s")


def _sc_degree(dst2d, wpw, npad):
    per_tile = npad // NS

    @functools.partial(
        pl.kernel,
        out_type=jax.ShapeDtypeStruct((NC, npad), jnp.float32),
        mesh=_MESH,
        compiler_params=pltpu.CompilerParams(use_tc_tiling_on_sc=False),
        scratch_types=[
            pltpu.VMEM_SHARED((npad,), jnp.float32),
            pltpu.VMEM((wpw, WIN), jnp.int32),
            pltpu.VMEM((WIN,), jnp.float32),
            pltpu.VMEM((per_tile,), jnp.float32),
            pltpu.SemaphoreType.DMA,
        ],
    )
    def deg_kernel(dst_hbm, out_hbm, accum, idx_v, ones_v, zb, sem):
        cid = lax.axis_index("c")
        sid = lax.axis_index("s")
        w = cid * NS + sid

        @pl.loop(0, per_tile // 16)
        def _(i):
            zb[pl.ds(i * 16, 16)] = jnp.zeros((16,), jnp.float32)

        @pl.loop(0, WIN // 16)
        def _(i):
            ones_v[pl.ds(i * 16, 16)] = jnp.full((16,), 1.0, jnp.float32)

        pltpu.sync_copy(zb, accum.at[pl.ds(sid * per_tile, per_tile)])
        pltpu.sync_copy(dst_hbm.at[pl.ds(w * wpw, wpw)], idx_v)
        plsc.subcore_barrier()

        @pl.loop(0, wpw)
        def _(j):
            pltpu.async_copy(ones_v, accum.at[idx_v.at[j]], sem, add=True)

        @pl.loop(0, wpw)
        def _(j):
            pltpu.make_async_copy(ones_v, accum.at[idx_v.at[j]], sem).wait()

        plsc.subcore_barrier()
        pltpu.sync_copy(accum.at[pl.ds(sid * per_tile, per_tile)],
                        out_hbm.at[cid, pl.ds(sid * per_tile, per_tile)])

    return deg_kernel(dst2d)


def _sc_aggregate(table, src2d, dst2d, wpw, npad):
    per_tile = npad // NS

    @functools.partial(
        pl.kernel,
        out_type=jax.ShapeDtypeStruct((NC, npad, HIDDEN), jnp.float32),
        mesh=_MESH,
        compiler_params=pltpu.CompilerParams(use_tc_tiling_on_sc=False),
        scratch_types=[
            pltpu.VMEM_SHARED((npad, HIDDEN), jnp.float32),
            pltpu.VMEM((wpw, WIN), jnp.int32),
            pltpu.VMEM((wpw, WIN), jnp.int32),
            pltpu.VMEM((2 * GRP, WIN, HIDDEN), jnp.float32),
            pltpu.VMEM((per_tile, HIDDEN), jnp.float32),
            pltpu.SemaphoreType.DMA,
            pltpu.SemaphoreType.DMA,
            pltpu.SemaphoreType.DMA,
            pltpu.SemaphoreType.DMA,
        ],
    )
    def agg_kernel(tab_hbm, src_hbm, dst_hbm, out_hbm,
                   accum, src_v, dst_v, bufs, zb, sg0, sg1, ss0, ss1):
        cid = lax.axis_index("c")
        sid = lax.axis_index("s")
        w = cid * NS + sid

        @pl.loop(0, per_tile)
        def _(i):
            zb[i, :] = jnp.zeros((HIDDEN,), jnp.float32)

        pltpu.sync_copy(zb, accum.at[pl.ds(sid * per_tile, per_tile)])
        pltpu.sync_copy(src_hbm.at[pl.ds(w * wpw, wpw)], src_v)
        pltpu.sync_copy(dst_hbm.at[pl.ds(w * wpw, wpw)], dst_v)
        plsc.subcore_barrier()

        sgs, sss = (sg0, sg1), (ss0, ss1)

        @pl.loop(0, wpw // (2 * GRP))
        def _(p):
            base = p * (2 * GRP)
            gd = []
            for grp in range(2):
                for b in range(GRP):
                    gd.append(pltpu.async_copy(
                        tab_hbm.at[src_v.at[base + grp * GRP + b]],
                        bufs.at[grp * GRP + b], sgs[grp]))
            sd = []
            for grp in range(2):
                for b in range(GRP):
                    gd[grp * GRP + b].wait()
                    sd.append(pltpu.async_copy(
                        bufs.at[grp * GRP + b],
                        accum.at[dst_v.at[base + grp * GRP + b]],
                        sss[grp], add=True))
            for d in sd:
                d.wait()

        plsc.subcore_barrier()
        pltpu.sync_copy(accum.at[pl.ds(sid * per_tile, per_tile)],
                        out_hbm.at[cid, pl.ds(sid * per_tile, per_tile)])

    return agg_kernel(table, src2d, dst2d)


def _tc_matmul(xp, W1):
    npad = xp.shape[0]

    def body(x_ref, w_ref, o_ref):
        o_ref[...] = lax.dot_general(
            x_ref[...], w_ref[...], (((1,), (1,)), ((), ())),
            preferred_element_type=jnp.float32)

    return pl.pallas_call(
        body,
        out_shape=jax.ShapeDtypeStruct((npad, HIDDEN), jnp.float32),
    )(xp, W1)


def _tc_scale(degT, h1):
    npad = h1.shape[0]

    def body(d_ref, h_ref, g_ref, dis_ref):
        deg = d_ref[:, 0:1] + d_ref[:, 1:2] + 1.0
        dis = lax.rsqrt(deg)
        dis_ref[...] = dis
        g_ref[...] = dis * h_ref[...]

    return pl.pallas_call(
        body,
        out_shape=(jax.ShapeDtypeStruct((npad, HIDDEN), jnp.float32),
                   jax.ShapeDtypeStruct((npad, 1), jnp.float32)),
    )(degT, h1)


def _tc_mid(P, g1, dis, b1row):
    npad = g1.shape[0]

    def body(p_ref, g_ref, dis_ref, b_ref, o_ref):
        agg = p_ref[0] + p_ref[1] + g_ref[...]
        pre = dis_ref[...] * agg + b_ref[...]
        z = jnp.where(pre >= 0, pre, 0.01 * pre)
        o_ref[...] = dis_ref[...] * z

    return pl.pallas_call(
        body,
        out_shape=jax.ShapeDtypeStruct((npad, HIDDEN), jnp.float32),
    )(P, g1, dis, b1row)


def _tc_final(Q, g2, dis, W2, b2row):
    npad = g2.shape[0]

    def body(q_ref, g_ref, dis_ref, w_ref, b_ref, o_ref):
        agg = dis_ref[...] * (q_ref[0] + q_ref[1] + g_ref[...])
        logits = lax.dot_general(
            agg, w_ref[...], (((1,), (1,)), ((), ())),
            preferred_element_type=jnp.float32) + b_ref[...]
        m = jnp.max(logits, axis=1, keepdims=True)
        s = jnp.sum(jnp.exp(logits - m), axis=1, keepdims=True)
        o_ref[...] = logits - (m + jnp.log(s))

    return pl.pallas_call(
        body,
        out_shape=jax.ShapeDtypeStruct((npad, N_OUT), jnp.float32),
    )(Q, g2, dis, W2, b2row)


def kernel(x, edge_index, W1, b1, W2, b2):
    n = x.shape[0]
    e = edge_index.shape[1]
    npad = -(-n // (NS * 16)) * (NS * 16)
    wpw = -(-e // (NW * WIN))
    wpw = -(-wpw // 8) * 8
    ep = wpw * NW * WIN

    ei = edge_index.astype(jnp.int32)
    pad = jnp.full((ep - e,), n, jnp.int32)
    src2d = jnp.concatenate([ei[0], pad]).reshape(NW * wpw, WIN)
    dst2d = jnp.concatenate([ei[1], pad]).reshape(NW * wpw, WIN)
    xp = jnp.pad(x, ((0, npad - n), (0, 0)))

    degh = _sc_degree(dst2d, wpw, npad)
    h1 = _tc_matmul(xp, W1)
    g1, dis = _tc_scale(degh.T, h1)
    P = _sc_aggregate(g1, src2d, dst2d, wpw, npad)
    g2 = _tc_mid(P, g1, dis, b1.reshape(1, HIDDEN))
    Q = _sc_aggregate(g2, src2d, dst2d, wpw, npad)
    out = _tc_final(Q, g2, dis, W2, b2.reshape(1, N_OUT))
    return out[:n]


# speedup vs baseline: 59.5726x; 1.4621x over previous
import functools

import jax
import jax.numpy as jnp
from jax import lax
from jax.experimental import pallas as pl
from jax.experimental.pallas import tpu as pltpu
from jax.experimental.pallas import tpu_sc as plsc

HIDDEN = 16
N_OUT = 2

NC = 2
NS = 16
NW = NC * NS
WIN = 128
GRP = 4
GPG = 2 * GRP

F0 = 0.683

_MESH = plsc.VectorSubcoreMesh(core_axis_name="c", subcore_axis_name="s")


def _split(nwin):
    g = nwin // GPG
    g0 = int(round(g * F0))
    g1 = g - g0
    per0, rem0 = divmod(g0, NS)
    per1, rem1 = divmod(g1, NS)
    return g0, per0, rem0, per1, rem1


def _my_groups(cid, sid, split):
    g0, per0, rem0, per1, rem1 = split
    gcnt = jnp.where(cid == 0, per0 + (sid < rem0), per1 + (sid < rem1))
    gbase = jnp.where(
        cid == 0,
        sid * per0 + jnp.minimum(sid, rem0),
        g0 + sid * per1 + jnp.minimum(sid, rem1))
    return gbase, gcnt


def _copy_my_windows(ei3, which, idx_v, cid, sid, gbase, split):
    g0, per0, rem0, per1, rem1 = split
    for pred, rows in (
            ((cid == 0) & (sid < rem0), (per0 + 1) * GPG),
            ((cid == 0) & (sid >= rem0), per0 * GPG),
            ((cid == 1) & (sid < rem1), (per1 + 1) * GPG),
            ((cid == 1) & (sid >= rem1), per1 * GPG),
    ):
        if rows > 0:
            @pl.when(pred)
            def _():
                pltpu.sync_copy(ei3.at[which, pl.ds(gbase * GPG, rows)],
                                idx_v.at[pl.ds(0, rows)])


def _sc_degree(ei3, nwin, npad):
    per_tile = npad // NS
    split = _split(nwin)
    max_rows = (max(split[1], split[3]) + 1) * GPG

    @functools.partial(
        pl.kernel,
        out_type=jax.ShapeDtypeStruct((NC, npad), jnp.float32),
        mesh=_MESH,
        compiler_params=pltpu.CompilerParams(use_tc_tiling_on_sc=False),
        scratch_types=[
            pltpu.VMEM_SHARED((npad,), jnp.float32),
            pltpu.VMEM((max_rows, WIN), jnp.int32),
            pltpu.VMEM((WIN,), jnp.float32),
            pltpu.VMEM((per_tile,), jnp.float32),
            pltpu.SemaphoreType.DMA,
        ],
    )
    def deg_kernel(ei_hbm, out_hbm, accum, idx_v, ones_v, zb, sem):
        cid = lax.axis_index("c")
        sid = lax.axis_index("s")
        gbase, gcnt = _my_groups(cid, sid, split)

        @pl.loop(0, per_tile // 16)
        def _(i):
            zb[pl.ds(i * 16, 16)] = jnp.zeros((16,), jnp.float32)

        @pl.loop(0, WIN // 16)
        def _(i):
            ones_v[pl.ds(i * 16, 16)] = jnp.full((16,), 1.0, jnp.float32)

        pltpu.sync_copy(zb, accum.at[pl.ds(sid * per_tile, per_tile)])
        _copy_my_windows(ei_hbm, 1, idx_v, cid, sid, gbase, split)
        plsc.subcore_barrier()

        @pl.loop(0, gcnt * GPG)
        def _(j):
            pltpu.async_copy(ones_v, accum.at[idx_v.at[j]], sem, add=True)

        @pl.loop(0, gcnt * GPG)
        def _(j):
            pltpu.make_async_copy(ones_v, accum.at[idx_v.at[j]], sem).wait()

        plsc.subcore_barrier()
        pltpu.sync_copy(accum.at[pl.ds(sid * per_tile, per_tile)],
                        out_hbm.at[cid, pl.ds(sid * per_tile, per_tile)])

    return deg_kernel(ei3)


def _sc_aggregate(table, ei3, nwin, npad):
    per_tile = npad // NS
    split = _split(nwin)
    max_rows = (max(split[1], split[3]) + 1) * GPG

    @functools.partial(
        pl.kernel,
        out_type=jax.ShapeDtypeStruct((NC, npad, HIDDEN), jnp.float32),
        mesh=_MESH,
        compiler_params=pltpu.CompilerParams(use_tc_tiling_on_sc=False),
        scratch_types=[
            pltpu.VMEM_SHARED((npad, HIDDEN), jnp.float32),
            pltpu.VMEM((max_rows, WIN), jnp.int32),
            pltpu.VMEM((max_rows, WIN), jnp.int32),
            pltpu.VMEM((2 * GRP, WIN, HIDDEN), jnp.float32),
            pltpu.VMEM((per_tile, HIDDEN), jnp.float32),
            pltpu.SemaphoreType.DMA,
            pltpu.SemaphoreType.DMA,
            pltpu.SemaphoreType.DMA,
            pltpu.SemaphoreType.DMA,
        ],
    )
    def agg_kernel(tab_hbm, ei_hbm, out_hbm,
                   accum, src_v, dst_v, bufs, zb, sg0, sg1, ss0, ss1):
        cid = lax.axis_index("c")
        sid = lax.axis_index("s")
        gbase, gcnt = _my_groups(cid, sid, split)

        @pl.loop(0, per_tile)
        def _(i):
            zb[i, :] = jnp.zeros((HIDDEN,), jnp.float32)

        pltpu.sync_copy(zb, accum.at[pl.ds(sid * per_tile, per_tile)])
        _copy_my_windows(ei_hbm, 0, src_v, cid, sid, gbase, split)
        _copy_my_windows(ei_hbm, 1, dst_v, cid, sid, gbase, split)
        plsc.subcore_barrier()

        sgs, sss = (sg0, sg1), (ss0, ss1)

        @pl.loop(0, gcnt)
        def _(p):
            base = p * GPG
            gd = []
            for grp in range(2):
                for b in range(GRP):
                    gd.append(pltpu.async_copy(
                        tab_hbm.at[src_v.at[base + grp * GRP + b]],
                        bufs.at[grp * GRP + b], sgs[grp]))
            sd = []
            for grp in range(2):
                for b in range(GRP):
                    gd[grp * GRP + b].wait()
                    sd.append(pltpu.async_copy(
                        bufs.at[grp * GRP + b],
                        accum.at[dst_v.at[base + grp * GRP + b]],
                        sss[grp], add=True))
            for d in sd:
                d.wait()

        plsc.subcore_barrier()
        pltpu.sync_copy(accum.at[pl.ds(sid * per_tile, per_tile)],
                        out_hbm.at[cid, pl.ds(sid * per_tile, per_tile)])

    return agg_kernel(table, ei3)


def _tc_matmul(xp, W1):
    npad = xp.shape[0]

    def body(x_ref, w_ref, o_ref):
        o_ref[...] = lax.dot_general(
            x_ref[...], w_ref[...], (((1,), (1,)), ((), ())),
            preferred_element_type=jnp.float32)

    return pl.pallas_call(
        body,
        out_shape=jax.ShapeDtypeStruct((npad, HIDDEN), jnp.float32),
    )(xp, W1)


def _tc_scale(degT, h1):
    npad = h1.shape[0]

    def body(d_ref, h_ref, g_ref, dis_ref):
        deg = d_ref[:, 0:1] + d_ref[:, 1:2] + 1.0
        dis = lax.rsqrt(deg)
        dis_ref[...] = dis
        g_ref[...] = dis * h_ref[...]

    return pl.pallas_call(
        body,
        out_shape=(jax.ShapeDtypeStruct((npad, HIDDEN), jnp.float32),
                   jax.ShapeDtypeStruct((npad, 1), jnp.float32)),
    )(degT, h1)


def _tc_mid(P, g1, dis, b1row):
    npad = g1.shape[0]

    def body(p_ref, g_ref, dis_ref, b_ref, o_ref):
        agg = p_ref[0] + p_ref[1] + g_ref[...]
        pre = dis_ref[...] * agg + b_ref[...]
        z = jnp.where(pre >= 0, pre, 0.01 * pre)
        o_ref[...] = dis_ref[...] * z

    return pl.pallas_call(
        body,
        out_shape=jax.ShapeDtypeStruct((npad, HIDDEN), jnp.float32),
    )(P, g1, dis, b1row)


def _tc_final(Q, g2, dis, W2, b2row):
    npad = g2.shape[0]

    def body(q_ref, g_ref, dis_ref, w_ref, b_ref, o_ref):
        agg = dis_ref[...] * (q_ref[0] + q_ref[1] + g_ref[...])
        logits = lax.dot_general(
            agg, w_ref[...], (((1,), (1,)), ((), ())),
            preferred_element_type=jnp.float32) + b_ref[...]
        m = jnp.max(logits, axis=1, keepdims=True)
        s = jnp.sum(jnp.exp(logits - m), axis=1, keepdims=True)
        o_ref[...] = logits - (m + jnp.log(s))

    return pl.pallas_call(
        body,
        out_shape=jax.ShapeDtypeStruct((npad, N_OUT), jnp.float32),
    )(Q, g2, dis, W2, b2row)


def kernel(x, edge_index, W1, b1, W2, b2):
    n = x.shape[0]
    e = edge_index.shape[1]
    npad = -(-n // (NS * 16)) * (NS * 16)
    nwin = -(-e // (WIN * GPG)) * GPG
    ep = nwin * WIN

    ei = edge_index.astype(jnp.int32)
    ei3 = jnp.pad(ei, ((0, 0), (0, ep - e)), constant_values=n).reshape(
        2, nwin, WIN)
    xp = jnp.pad(x, ((0, npad - n), (0, 0)))

    degh = _sc_degree(ei3, nwin, npad)
    h1 = _tc_matmul(xp, W1)
    g1, dis = _tc_scale(degh.T, h1)
    P = _sc_aggregate(g1, ei3, nwin, npad)
    g2 = _tc_mid(P, g1, dis, b1.reshape(1, HIDDEN))
    Q = _sc_aggregate(g2, ei3, nwin, npad)
    out = _tc_final(Q, g2, dis, W2, b2.reshape(1, N_OUT))
    return out[:n]


# speedup vs baseline: 62.0343x; 1.0413x over previous
import functools

import jax
import jax.numpy as jnp
from jax import lax
from jax.experimental import pallas as pl
from jax.experimental.pallas import tpu as pltpu
from jax.experimental.pallas import tpu_sc as plsc

HIDDEN = 16
N_OUT = 2

NC = 2
NS = 16
NW = NC * NS
WIN = 128
GRP = 4
GPG = 2 * GRP

F0 = 0.57

_MESH = plsc.VectorSubcoreMesh(core_axis_name="c", subcore_axis_name="s")


def _split(nwin):
    g = nwin // GPG
    g0 = int(round(g * F0))
    g1 = g - g0
    per0, rem0 = divmod(g0, NS)
    per1, rem1 = divmod(g1, NS)
    return g0, per0, rem0, per1, rem1


def _my_groups(cid, sid, split):
    g0, per0, rem0, per1, rem1 = split
    gcnt = jnp.where(cid == 0, per0 + (sid < rem0), per1 + (sid < rem1))
    gbase = jnp.where(
        cid == 0,
        sid * per0 + jnp.minimum(sid, rem0),
        g0 + sid * per1 + jnp.minimum(sid, rem1))
    return gbase, gcnt


def _copy_my_windows(ei3, which, idx_v, cid, sid, gbase, split):
    g0, per0, rem0, per1, rem1 = split
    for pred, rows in (
            ((cid == 0) & (sid < rem0), (per0 + 1) * GPG),
            ((cid == 0) & (sid >= rem0), per0 * GPG),
            ((cid == 1) & (sid < rem1), (per1 + 1) * GPG),
            ((cid == 1) & (sid >= rem1), per1 * GPG),
    ):
        if rows > 0:
            @pl.when(pred)
            def _():
                pltpu.sync_copy(ei3.at[which, pl.ds(gbase * GPG, rows)],
                                idx_v.at[pl.ds(0, rows)])


def _sc_degree(ei3, nwin, npad):
    per_tile = npad // NS
    split = _split(nwin)
    max_rows = (max(split[1], split[3]) + 1) * GPG

    @functools.partial(
        pl.kernel,
        out_type=jax.ShapeDtypeStruct((NC, npad), jnp.float32),
        mesh=_MESH,
        compiler_params=pltpu.CompilerParams(use_tc_tiling_on_sc=False),
        scratch_types=[
            pltpu.VMEM_SHARED((npad,), jnp.float32),
            pltpu.VMEM((max_rows, WIN), jnp.int32),
            pltpu.VMEM((WIN,), jnp.float32),
            pltpu.VMEM((per_tile,), jnp.float32),
            pltpu.SemaphoreType.DMA,
        ],
    )
    def deg_kernel(ei_hbm, out_hbm, accum, idx_v, ones_v, zb, sem):
        cid = lax.axis_index("c")
        sid = lax.axis_index("s")
        gbase, gcnt = _my_groups(cid, sid, split)

        @pl.loop(0, per_tile // 16)
        def _(i):
            zb[pl.ds(i * 16, 16)] = jnp.zeros((16,), jnp.float32)

        @pl.loop(0, WIN // 16)
        def _(i):
            ones_v[pl.ds(i * 16, 16)] = jnp.full((16,), 1.0, jnp.float32)

        pltpu.sync_copy(zb, accum.at[pl.ds(sid * per_tile, per_tile)])
        _copy_my_windows(ei_hbm, 1, idx_v, cid, sid, gbase, split)
        plsc.subcore_barrier()

        @pl.loop(0, gcnt * GPG)
        def _(j):
            pltpu.async_copy(ones_v, accum.at[idx_v.at[j]], sem, add=True)

        @pl.loop(0, gcnt * GPG)
        def _(j):
            pltpu.make_async_copy(ones_v, accum.at[idx_v.at[j]], sem).wait()

        plsc.subcore_barrier()
        pltpu.sync_copy(accum.at[pl.ds(sid * per_tile, per_tile)],
                        out_hbm.at[cid, pl.ds(sid * per_tile, per_tile)])

    return deg_kernel(ei3)


def _sc_aggregate(table, ei3, nwin, npad):
    per_tile = npad // NS
    split = _split(nwin)
    max_rows = (max(split[1], split[3]) + 1) * GPG

    @functools.partial(
        pl.kernel,
        out_type=jax.ShapeDtypeStruct((NC, npad, HIDDEN), jnp.float32),
        mesh=_MESH,
        compiler_params=pltpu.CompilerParams(use_tc_tiling_on_sc=False),
        scratch_types=[
            pltpu.VMEM_SHARED((npad, HIDDEN), jnp.float32),
            pltpu.VMEM((max_rows, WIN), jnp.int32),
            pltpu.VMEM((max_rows, WIN), jnp.int32),
            pltpu.VMEM((2 * GRP, WIN, HIDDEN), jnp.float32),
            pltpu.VMEM((per_tile, HIDDEN), jnp.float32),
            pltpu.SemaphoreType.DMA,
            pltpu.SemaphoreType.DMA,
            pltpu.SemaphoreType.DMA,
            pltpu.SemaphoreType.DMA,
        ],
    )
    def agg_kernel(tab_hbm, ei_hbm, out_hbm,
                   accum, src_v, dst_v, bufs, zb, sg0, sg1, ss0, ss1):
        cid = lax.axis_index("c")
        sid = lax.axis_index("s")
        gbase, gcnt = _my_groups(cid, sid, split)

        @pl.loop(0, per_tile)
        def _(i):
            zb[i, :] = jnp.zeros((HIDDEN,), jnp.float32)

        pltpu.sync_copy(zb, accum.at[pl.ds(sid * per_tile, per_tile)])
        _copy_my_windows(ei_hbm, 0, src_v, cid, sid, gbase, split)
        _copy_my_windows(ei_hbm, 1, dst_v, cid, sid, gbase, split)
        plsc.subcore_barrier()

        sgs, sss = (sg0, sg1), (ss0, ss1)

        @pl.loop(0, gcnt)
        def _(p):
            base = p * GPG
            gd = []
            for grp in range(2):
                for b in range(GRP):
                    gd.append(pltpu.async_copy(
                        tab_hbm.at[src_v.at[base + grp * GRP + b]],
                        bufs.at[grp * GRP + b], sgs[grp]))
            sd = []
            for grp in range(2):
                for b in range(GRP):
                    gd[grp * GRP + b].wait()
                    sd.append(pltpu.async_copy(
                        bufs.at[grp * GRP + b],
                        accum.at[dst_v.at[base + grp * GRP + b]],
                        sss[grp], add=True))
            for d in sd:
                d.wait()

        plsc.subcore_barrier()
        pltpu.sync_copy(accum.at[pl.ds(sid * per_tile, per_tile)],
                        out_hbm.at[cid, pl.ds(sid * per_tile, per_tile)])

    return agg_kernel(table, ei3)


def _tc_matmul(xp, W1):
    npad = xp.shape[0]

    def body(x_ref, w_ref, o_ref):
        o_ref[...] = lax.dot_general(
            x_ref[...], w_ref[...], (((1,), (1,)), ((), ())),
            preferred_element_type=jnp.float32)

    return pl.pallas_call(
        body,
        out_shape=jax.ShapeDtypeStruct((npad, HIDDEN), jnp.float32),
    )(xp, W1)


def _tc_scale(degT, h1):
    npad = h1.shape[0]

    def body(d_ref, h_ref, g_ref, dis_ref):
        deg = d_ref[:, 0:1] + d_ref[:, 1:2] + 1.0
        dis = lax.rsqrt(deg)
        dis_ref[...] = dis
        g_ref[...] = dis * h_ref[...]

    return pl.pallas_call(
        body,
        out_shape=(jax.ShapeDtypeStruct((npad, HIDDEN), jnp.float32),
                   jax.ShapeDtypeStruct((npad, 1), jnp.float32)),
    )(degT, h1)


def _tc_mid(P, g1, dis, b1row):
    npad = g1.shape[0]

    def body(p_ref, g_ref, dis_ref, b_ref, o_ref):
        agg = p_ref[0] + p_ref[1] + g_ref[...]
        pre = dis_ref[...] * agg + b_ref[...]
        z = jnp.where(pre >= 0, pre, 0.01 * pre)
        o_ref[...] = dis_ref[...] * z

    return pl.pallas_call(
        body,
        out_shape=jax.ShapeDtypeStruct((npad, HIDDEN), jnp.float32),
    )(P, g1, dis, b1row)


def _tc_final(Q, g2, dis, W2, b2row):
    npad = g2.shape[0]

    def body(q_ref, g_ref, dis_ref, w_ref, b_ref, o_ref):
        agg = dis_ref[...] * (q_ref[0] + q_ref[1] + g_ref[...])
        logits = lax.dot_general(
            agg, w_ref[...], (((1,), (1,)), ((), ())),
            preferred_element_type=jnp.float32) + b_ref[...]
        m = jnp.max(logits, axis=1, keepdims=True)
        s = jnp.sum(jnp.exp(logits - m), axis=1, keepdims=True)
        o_ref[...] = logits - (m + jnp.log(s))

    return pl.pallas_call(
        body,
        out_shape=jax.ShapeDtypeStruct((npad, N_OUT), jnp.float32),
    )(Q, g2, dis, W2, b2row)


def kernel(x, edge_index, W1, b1, W2, b2):
    n = x.shape[0]
    e = edge_index.shape[1]
    npad = -(-n // (NS * 16)) * (NS * 16)
    nwin = -(-e // (WIN * GPG)) * GPG
    ep = nwin * WIN

    ei = edge_index.astype(jnp.int32)
    ei3 = jnp.pad(ei, ((0, 0), (0, ep - e)), constant_values=n).reshape(
        2, nwin, WIN)
    xp = jnp.pad(x, ((0, npad - n), (0, 0)))

    degh = _sc_degree(ei3, nwin, npad)
    h1 = _tc_matmul(xp, W1)
    g1, dis = _tc_scale(degh.T, h1)
    P = _sc_aggregate(g1, ei3, nwin, npad)
    g2 = _tc_mid(P, g1, dis, b1.reshape(1, HIDDEN))
    Q = _sc_aggregate(g2, ei3, nwin, npad)
    out = _tc_final(Q, g2, dis, W2, b2.reshape(1, N_OUT))
    return out[:n]


# speedup vs baseline: 69.6471x; 1.1227x over previous
import functools

import jax
import jax.numpy as jnp
from jax import lax
from jax.experimental import pallas as pl
from jax.experimental.pallas import tpu as pltpu
from jax.experimental.pallas import tpu_sc as plsc

HIDDEN = 16
N_OUT = 2

NC = 2
NS = 16
NW = NC * NS
WIN = 128
GRP = 4
GPG = 2 * GRP

F0 = 0.57

_MESH = plsc.VectorSubcoreMesh(core_axis_name="c", subcore_axis_name="s")


def _split(nwin):
    g = nwin // GPG
    g0 = int(round(g * F0))
    g1 = g - g0
    per0, rem0 = divmod(g0, NS)
    per1, rem1 = divmod(g1, NS)
    return g0, per0, rem0, per1, rem1


def _my_groups(cid, sid, split):
    g0, per0, rem0, per1, rem1 = split
    gcnt = jnp.where(cid == 0, per0 + (sid < rem0), per1 + (sid < rem1))
    gbase = jnp.where(
        cid == 0,
        sid * per0 + jnp.minimum(sid, rem0),
        g0 + sid * per1 + jnp.minimum(sid, rem1))
    return gbase, gcnt


def _copy_my_windows(ei3, which, idx_v, cid, sid, gbase, split):
    g0, per0, rem0, per1, rem1 = split
    for pred, rows in (
            ((cid == 0) & (sid < rem0), (per0 + 1) * GPG),
            ((cid == 0) & (sid >= rem0), per0 * GPG),
            ((cid == 1) & (sid < rem1), (per1 + 1) * GPG),
            ((cid == 1) & (sid >= rem1), per1 * GPG),
    ):
        if rows > 0:
            @pl.when(pred)
            def _():
                pltpu.sync_copy(ei3.at[which, pl.ds(gbase * GPG, rows)],
                                idx_v.at[pl.ds(0, rows)])


def _sc_degree(ei3, nwin, npad):
    per_tile = npad // NS
    split = _split(nwin)
    max_rows = (max(split[1], split[3]) + 1) * GPG

    @functools.partial(
        pl.kernel,
        out_type=jax.ShapeDtypeStruct((NC, npad), jnp.float32),
        mesh=_MESH,
        compiler_params=pltpu.CompilerParams(use_tc_tiling_on_sc=False),
        scratch_types=[
            pltpu.VMEM_SHARED((npad,), jnp.float32),
            pltpu.VMEM((max_rows, WIN), jnp.int32),
            pltpu.VMEM((WIN,), jnp.float32),
            pltpu.VMEM((per_tile,), jnp.float32),
            pltpu.SemaphoreType.DMA,
        ],
    )
    def deg_kernel(ei_hbm, out_hbm, accum, idx_v, ones_v, zb, sem):
        cid = lax.axis_index("c")
        sid = lax.axis_index("s")
        gbase, gcnt = _my_groups(cid, sid, split)

        @pl.loop(0, per_tile // 16)
        def _(i):
            zb[pl.ds(i * 16, 16)] = jnp.zeros((16,), jnp.float32)

        @pl.loop(0, WIN // 16)
        def _(i):
            ones_v[pl.ds(i * 16, 16)] = jnp.full((16,), 1.0, jnp.float32)

        pltpu.sync_copy(zb, accum.at[pl.ds(sid * per_tile, per_tile)])
        _copy_my_windows(ei_hbm, 1, idx_v, cid, sid, gbase, split)
        plsc.subcore_barrier()

        @pl.loop(0, gcnt * GPG)
        def _(j):
            pltpu.async_copy(ones_v, accum.at[idx_v.at[j]], sem, add=True)

        @pl.loop(0, gcnt * GPG)
        def _(j):
            pltpu.make_async_copy(ones_v, accum.at[idx_v.at[j]], sem).wait()

        plsc.subcore_barrier()
        pltpu.sync_copy(accum.at[pl.ds(sid * per_tile, per_tile)],
                        out_hbm.at[cid, pl.ds(sid * per_tile, per_tile)])

    return deg_kernel(ei3)


def _sc_aggregate(table, ei3, nwin, npad):
    per_tile = npad // NS
    split = _split(nwin)
    max_rows = (max(split[1], split[3]) + 1) * GPG

    @functools.partial(
        pl.kernel,
        out_type=jax.ShapeDtypeStruct((NC, npad, HIDDEN), jnp.float32),
        mesh=_MESH,
        compiler_params=pltpu.CompilerParams(use_tc_tiling_on_sc=False),
        scratch_types=[
            pltpu.VMEM_SHARED((npad, HIDDEN), jnp.float32),
            pltpu.VMEM((max_rows, WIN), jnp.int32),
            pltpu.VMEM((max_rows, WIN), jnp.int32),
            pltpu.VMEM((2 * GRP, WIN, HIDDEN), jnp.float32),
            pltpu.VMEM((per_tile, HIDDEN), jnp.float32),
            pltpu.SemaphoreType.DMA,
            pltpu.SemaphoreType.DMA,
            pltpu.SemaphoreType.DMA,
            pltpu.SemaphoreType.DMA,
        ],
    )
    def agg_kernel(tab_hbm, ei_hbm, out_hbm,
                   accum, src_v, dst_v, bufs, zb, sg0, sg1, ss0, ss1):
        cid = lax.axis_index("c")
        sid = lax.axis_index("s")
        gbase, gcnt = _my_groups(cid, sid, split)

        @pl.loop(0, per_tile)
        def _(i):
            zb[i, :] = jnp.zeros((HIDDEN,), jnp.float32)

        pltpu.sync_copy(zb, accum.at[pl.ds(sid * per_tile, per_tile)])
        _copy_my_windows(ei_hbm, 0, src_v, cid, sid, gbase, split)
        _copy_my_windows(ei_hbm, 1, dst_v, cid, sid, gbase, split)
        plsc.subcore_barrier()

        sgs, sss = (sg0, sg1), (ss0, ss1)

        @pl.loop(0, gcnt)
        def _(p):
            base = p * GPG
            gd = []
            for grp in range(2):
                for b in range(GRP):
                    gd.append(pltpu.async_copy(
                        tab_hbm.at[src_v.at[base + grp * GRP + b]],
                        bufs.at[grp * GRP + b], sgs[grp]))
            sd = []
            for grp in range(2):
                for b in range(GRP):
                    gd[grp * GRP + b].wait()
                    sd.append(pltpu.async_copy(
                        bufs.at[grp * GRP + b],
                        accum.at[dst_v.at[base + grp * GRP + b]],
                        sss[grp], add=True))
            for d in sd:
                d.wait()

        plsc.subcore_barrier()
        pltpu.sync_copy(accum.at[pl.ds(sid * per_tile, per_tile)],
                        out_hbm.at[cid, pl.ds(sid * per_tile, per_tile)])

    return agg_kernel(table, ei3)


def _tc_matmul(xT, W1):
    npad = xT.shape[1]

    def body(w_ref, x_ref, o_ref):
        o_ref[...] = lax.dot_general(
            w_ref[...], x_ref[...], (((1,), (0,)), ((), ())),
            preferred_element_type=jnp.float32)

    return pl.pallas_call(
        body,
        out_shape=jax.ShapeDtypeStruct((HIDDEN, npad), jnp.float32),
    )(W1, xT)


def _tc_scale(degh, h1T):
    npad = h1T.shape[1]

    def body(d_ref, h_ref, g_ref, dis_ref):
        deg = d_ref[0:1, :] + d_ref[1:2, :] + 1.0
        dis = lax.rsqrt(deg)
        dis_ref[...] = dis
        g_ref[...] = dis * h_ref[...]

    return pl.pallas_call(
        body,
        out_shape=(jax.ShapeDtypeStruct((HIDDEN, npad), jnp.float32),
                   jax.ShapeDtypeStruct((1, npad), jnp.float32)),
    )(degh, h1T)


def _tc_mid(PT, g1T, dis, b1col):
    npad = g1T.shape[1]

    def body(p_ref, g_ref, dis_ref, b_ref, o_ref):
        agg = p_ref[0] + p_ref[1] + g_ref[...]
        pre = dis_ref[...] * agg + b_ref[...]
        z = jnp.where(pre >= 0, pre, 0.01 * pre)
        o_ref[...] = dis_ref[...] * z

    return pl.pallas_call(
        body,
        out_shape=jax.ShapeDtypeStruct((HIDDEN, npad), jnp.float32),
    )(PT, g1T, dis, b1col)


def _tc_final(QT, g2T, dis, W2, b2col):
    npad = g2T.shape[1]

    def body(q_ref, g_ref, dis_ref, w_ref, b_ref, o_ref):
        agg = dis_ref[...] * (q_ref[0] + q_ref[1] + g_ref[...])
        logits = lax.dot_general(
            w_ref[...], agg, (((1,), (0,)), ((), ())),
            preferred_element_type=jnp.float32) + b_ref[...]
        m = jnp.max(logits, axis=0, keepdims=True)
        s = jnp.sum(jnp.exp(logits - m), axis=0, keepdims=True)
        o_ref[...] = logits - (m + jnp.log(s))

    return pl.pallas_call(
        body,
        out_shape=jax.ShapeDtypeStruct((N_OUT, npad), jnp.float32),
    )(QT, g2T, dis, W2, b2col)


def kernel(x, edge_index, W1, b1, W2, b2):
    n = x.shape[0]
    e = edge_index.shape[1]
    npad = -(-n // (NS * 16)) * (NS * 16)
    nwin = -(-e // (WIN * GPG)) * GPG
    ep = nwin * WIN

    ei = edge_index.astype(jnp.int32)
    ei3 = jnp.pad(ei, ((0, 0), (0, ep - e)), constant_values=n).reshape(
        2, nwin, WIN)
    xT = jnp.pad(x.T, ((0, 0), (0, npad - n)))

    degh = _sc_degree(ei3, nwin, npad)
    h1T = _tc_matmul(xT, W1)
    g1T, dis = _tc_scale(degh, h1T)
    P = _sc_aggregate(g1T.T, ei3, nwin, npad)
    g2T = _tc_mid(P.transpose(0, 2, 1), g1T, dis, b1.reshape(HIDDEN, 1))
    Q = _sc_aggregate(g2T.T, ei3, nwin, npad)
    outT = _tc_final(Q.transpose(0, 2, 1), g2T, dis, W2, b2.reshape(N_OUT, 1))
    return outT.T[:n]


# speedup vs baseline: 74.3296x; 1.0672x over previous
import functools

import jax
import jax.numpy as jnp
from jax import lax
from jax.experimental import pallas as pl
from jax.experimental.pallas import tpu as pltpu
from jax.experimental.pallas import tpu_sc as plsc

HIDDEN = 16
N_OUT = 2

NC = 2
NS = 16
NW = NC * NS
WIN = 128
GRP = 4
GPG = 2 * GRP

F0 = 0.57

_MESH = plsc.VectorSubcoreMesh(core_axis_name="c", subcore_axis_name="s")


def _split(nwin):
    g = nwin // GPG
    g0 = int(round(g * F0))
    g1 = g - g0
    per0, rem0 = divmod(g0, NS)
    per1, rem1 = divmod(g1, NS)
    return g0, per0, rem0, per1, rem1


def _my_groups(cid, sid, split):
    g0, per0, rem0, per1, rem1 = split
    gcnt = jnp.where(cid == 0, per0 + (sid < rem0), per1 + (sid < rem1))
    gbase = jnp.where(
        cid == 0,
        sid * per0 + jnp.minimum(sid, rem0),
        g0 + sid * per1 + jnp.minimum(sid, rem1))
    return gbase, gcnt


def _copy_my_windows(ei3, which, idx_v, cid, sid, gbase, split):
    g0, per0, rem0, per1, rem1 = split
    for pred, rows in (
            ((cid == 0) & (sid < rem0), (per0 + 1) * GPG),
            ((cid == 0) & (sid >= rem0), per0 * GPG),
            ((cid == 1) & (sid < rem1), (per1 + 1) * GPG),
            ((cid == 1) & (sid >= rem1), per1 * GPG),
    ):
        if rows > 0:
            @pl.when(pred)
            def _():
                pltpu.sync_copy(ei3.at[which, pl.ds(gbase * GPG, rows)],
                                idx_v.at[pl.ds(0, rows)])


def _sc_degree(ei3, nwin, npad):
    per_tile = npad // NS
    split = _split(nwin)
    max_rows = (max(split[1], split[3]) + 1) * GPG

    @functools.partial(
        pl.kernel,
        out_type=jax.ShapeDtypeStruct((NC, npad), jnp.float32),
        mesh=_MESH,
        compiler_params=pltpu.CompilerParams(use_tc_tiling_on_sc=False),
        scratch_types=[
            pltpu.VMEM_SHARED((npad,), jnp.float32),
            pltpu.VMEM((max_rows, WIN), jnp.int32),
            pltpu.VMEM((WIN,), jnp.float32),
            pltpu.VMEM((per_tile,), jnp.float32),
            pltpu.SemaphoreType.DMA,
        ],
    )
    def deg_kernel(ei_hbm, out_hbm, accum, idx_v, ones_v, zb, sem):
        cid = lax.axis_index("c")
        sid = lax.axis_index("s")
        gbase, gcnt = _my_groups(cid, sid, split)

        @pl.loop(0, per_tile // 16)
        def _(i):
            zb[pl.ds(i * 16, 16)] = jnp.zeros((16,), jnp.float32)

        @pl.loop(0, WIN // 16)
        def _(i):
            ones_v[pl.ds(i * 16, 16)] = jnp.full((16,), 1.0, jnp.float32)

        pltpu.sync_copy(zb, accum.at[pl.ds(sid * per_tile, per_tile)])
        _copy_my_windows(ei_hbm, 1, idx_v, cid, sid, gbase, split)
        plsc.subcore_barrier()

        @pl.loop(0, gcnt * GPG)
        def _(j):
            pltpu.async_copy(ones_v, accum.at[idx_v.at[j]], sem, add=True)

        @pl.loop(0, gcnt * GPG)
        def _(j):
            pltpu.make_async_copy(ones_v, accum.at[idx_v.at[j]], sem).wait()

        plsc.subcore_barrier()
        pltpu.sync_copy(accum.at[pl.ds(sid * per_tile, per_tile)],
                        out_hbm.at[cid, pl.ds(sid * per_tile, per_tile)])

    return deg_kernel(ei3)


def _sc_aggregate(table, ei3, nwin, npad):
    per_tile = npad // NS
    split = _split(nwin)
    max_rows = (max(split[1], split[3]) + 1) * GPG

    @functools.partial(
        pl.kernel,
        out_type=jax.ShapeDtypeStruct((NC, HIDDEN, npad), jnp.float32),
        mesh=_MESH,
        compiler_params=pltpu.CompilerParams(use_tc_tiling_on_sc=False,
                                             needs_layout_passes=False),
        scratch_types=[
            pltpu.VMEM_SHARED((npad, HIDDEN), jnp.float32),
            pltpu.VMEM((max_rows, WIN), jnp.int32),
            pltpu.VMEM((max_rows, WIN), jnp.int32),
            pltpu.VMEM((2 * GRP, WIN, HIDDEN), jnp.float32),
            pltpu.VMEM((per_tile, HIDDEN), jnp.float32),
            pltpu.VMEM((HIDDEN * per_tile,), jnp.float32),
            pltpu.SemaphoreType.DMA,
            pltpu.SemaphoreType.DMA,
            pltpu.SemaphoreType.DMA,
            pltpu.SemaphoreType.DMA,
        ],
    )
    def agg_kernel(tab_hbm, ei_hbm, out_hbm,
                   accum, src_v, dst_v, bufs, zb, tbuf, sg0, sg1, ss0, ss1):
        cid = lax.axis_index("c")
        sid = lax.axis_index("s")
        gbase, gcnt = _my_groups(cid, sid, split)

        @pl.loop(0, per_tile)
        def _(i):
            zb[i, :] = jnp.zeros((HIDDEN,), jnp.float32)

        pltpu.sync_copy(zb, accum.at[pl.ds(sid * per_tile, per_tile)])
        _copy_my_windows(ei_hbm, 0, src_v, cid, sid, gbase, split)
        _copy_my_windows(ei_hbm, 1, dst_v, cid, sid, gbase, split)
        plsc.subcore_barrier()

        sgs, sss = (sg0, sg1), (ss0, ss1)

        @pl.loop(0, gcnt)
        def _(p):
            base = p * GPG
            gd = []
            for grp in range(2):
                for b in range(GRP):
                    gd.append(pltpu.async_copy(
                        tab_hbm.at[src_v.at[base + grp * GRP + b]],
                        bufs.at[grp * GRP + b], sgs[grp]))
            sd = []
            for grp in range(2):
                for b in range(GRP):
                    gd[grp * GRP + b].wait()
                    sd.append(pltpu.async_copy(
                        bufs.at[grp * GRP + b],
                        accum.at[dst_v.at[base + grp * GRP + b]],
                        sss[grp], add=True))
            for d in sd:
                d.wait()

        plsc.subcore_barrier()

        pltpu.sync_copy(accum.at[pl.ds(sid * per_tile, per_tile)], zb)
        col0 = lax.iota(jnp.int32, 16) * per_tile

        @pl.loop(0, per_tile)
        def _(i):
            plsc.store_scatter(tbuf, (col0 + i,), zb[i, :])

        for h in range(HIDDEN):
            pltpu.sync_copy(tbuf.at[pl.ds(h * per_tile, per_tile)],
                            out_hbm.at[cid, h, pl.ds(sid * per_tile, per_tile)])

    return agg_kernel(table, ei3)


def _tc_matmul(xT, W1):
    npad = xT.shape[1]

    def body(w_ref, x_ref, o_ref):
        o_ref[...] = lax.dot_general(
            w_ref[...], x_ref[...], (((1,), (0,)), ((), ())),
            preferred_element_type=jnp.float32)

    return pl.pallas_call(
        body,
        out_shape=jax.ShapeDtypeStruct((HIDDEN, npad), jnp.float32),
    )(W1, xT)


def _tc_scale(degh, h1T):
    npad = h1T.shape[1]

    def body(d_ref, h_ref, g_ref, dis_ref):
        deg = d_ref[0:1, :] + d_ref[1:2, :] + 1.0
        dis = lax.rsqrt(deg)
        dis_ref[...] = dis
        g_ref[...] = dis * h_ref[...]

    return pl.pallas_call(
        body,
        out_shape=(jax.ShapeDtypeStruct((HIDDEN, npad), jnp.float32),
                   jax.ShapeDtypeStruct((1, npad), jnp.float32)),
    )(degh, h1T)


def _tc_mid(PT, g1T, dis, b1col):
    npad = g1T.shape[1]

    def body(p_ref, g_ref, dis_ref, b_ref, o_ref):
        agg = p_ref[0] + p_ref[1] + g_ref[...]
        pre = dis_ref[...] * agg + b_ref[...]
        z = jnp.where(pre >= 0, pre, 0.01 * pre)
        o_ref[...] = dis_ref[...] * z

    return pl.pallas_call(
        body,
        out_shape=jax.ShapeDtypeStruct((HIDDEN, npad), jnp.float32),
    )(PT, g1T, dis, b1col)


def _tc_final(QT, g2T, dis, W2, b2col):
    npad = g2T.shape[1]

    def body(q_ref, g_ref, dis_ref, w_ref, b_ref, o_ref):
        agg = dis_ref[...] * (q_ref[0] + q_ref[1] + g_ref[...])
        logits = lax.dot_general(
            w_ref[...], agg, (((1,), (0,)), ((), ())),
            preferred_element_type=jnp.float32) + b_ref[...]
        m = jnp.max(logits, axis=0, keepdims=True)
        s = jnp.sum(jnp.exp(logits - m), axis=0, keepdims=True)
        o_ref[...] = logits - (m + jnp.log(s))

    return pl.pallas_call(
        body,
        out_shape=jax.ShapeDtypeStruct((N_OUT, npad), jnp.float32),
    )(QT, g2T, dis, W2, b2col)


def kernel(x, edge_index, W1, b1, W2, b2):
    n = x.shape[0]
    e = edge_index.shape[1]
    npad = -(-n // (NS * 16)) * (NS * 16)
    nwin = -(-e // (WIN * GPG)) * GPG
    ep = nwin * WIN

    ei = edge_index.astype(jnp.int32)
    ei3 = jnp.pad(ei, ((0, 0), (0, ep - e)), constant_values=n).reshape(
        2, nwin, WIN)
    xT = jnp.pad(x.T, ((0, 0), (0, npad - n)))

    degh = _sc_degree(ei3, nwin, npad)
    h1T = _tc_matmul(xT, W1)
    g1T, dis = _tc_scale(degh, h1T)
    P = _sc_aggregate(g1T.T, ei3, nwin, npad)
    g2T = _tc_mid(P, g1T, dis, b1.reshape(HIDDEN, 1))
    Q = _sc_aggregate(g2T.T, ei3, nwin, npad)
    outT = _tc_final(Q, g2T, dis, W2, b2.reshape(N_OUT, 1))
    return outT.T[:n]


# speedup vs baseline: 79.4662x; 1.0691x over previous
import functools

import jax
import jax.numpy as jnp
from jax import lax
from jax.experimental import pallas as pl
from jax.experimental.pallas import tpu as pltpu
from jax.experimental.pallas import tpu_sc as plsc

HIDDEN = 16
N_OUT = 2

NC = 2
NS = 16
NW = NC * NS
WIN = 128
SETW = 4
NSETS = 3
UNITW = SETW * NSETS

F0 = 0.57

_MESH = plsc.VectorSubcoreMesh(core_axis_name="c", subcore_axis_name="s")


def _split(nwin):
    g = nwin // UNITW
    g0 = int(round(g * F0))
    g1 = g - g0
    per0, rem0 = divmod(g0, NS)
    per1, rem1 = divmod(g1, NS)
    return g0, per0, rem0, per1, rem1


def _my_units(cid, sid, split):
    g0, per0, rem0, per1, rem1 = split
    ucnt = jnp.where(cid == 0, per0 + (sid < rem0), per1 + (sid < rem1))
    ubase = jnp.where(
        cid == 0,
        sid * per0 + jnp.minimum(sid, rem0),
        g0 + sid * per1 + jnp.minimum(sid, rem1))
    return ubase, ucnt


def _copy_my_windows(ei3, which, idx_v, cid, sid, ubase, split):
    g0, per0, rem0, per1, rem1 = split
    for pred, rows in (
            ((cid == 0) & (sid < rem0), (per0 + 1) * UNITW),
            ((cid == 0) & (sid >= rem0), per0 * UNITW),
            ((cid == 1) & (sid < rem1), (per1 + 1) * UNITW),
            ((cid == 1) & (sid >= rem1), per1 * UNITW),
    ):
        if rows > 0:
            @pl.when(pred)
            def _():
                pltpu.sync_copy(ei3.at[which, pl.ds(ubase * UNITW, rows)],
                                idx_v.at[pl.ds(0, rows)])


def _sc_degree(ei3, nwin, npad):
    per_tile = npad // NS
    split = _split(nwin)
    max_rows = (max(split[1], split[3]) + 1) * UNITW

    @functools.partial(
        pl.kernel,
        out_type=jax.ShapeDtypeStruct((NC, npad), jnp.float32),
        mesh=_MESH,
        compiler_params=pltpu.CompilerParams(use_tc_tiling_on_sc=False),
        scratch_types=[
            pltpu.VMEM_SHARED((npad,), jnp.float32),
            pltpu.VMEM((max_rows, WIN), jnp.int32),
            pltpu.VMEM((WIN,), jnp.float32),
            pltpu.VMEM((per_tile,), jnp.float32),
            pltpu.SemaphoreType.DMA,
        ],
    )
    def deg_kernel(ei_hbm, out_hbm, accum, idx_v, ones_v, zb, sem):
        cid = lax.axis_index("c")
        sid = lax.axis_index("s")
        ubase, ucnt = _my_units(cid, sid, split)

        @pl.loop(0, per_tile // 16)
        def _(i):
            zb[pl.ds(i * 16, 16)] = jnp.zeros((16,), jnp.float32)

        @pl.loop(0, WIN // 16)
        def _(i):
            ones_v[pl.ds(i * 16, 16)] = jnp.full((16,), 1.0, jnp.float32)

        pltpu.sync_copy(zb, accum.at[pl.ds(sid * per_tile, per_tile)])
        _copy_my_windows(ei_hbm, 1, idx_v, cid, sid, ubase, split)
        plsc.subcore_barrier()

        @pl.loop(0, ucnt * UNITW)
        def _(j):
            pltpu.async_copy(ones_v, accum.at[idx_v.at[j]], sem, add=True)

        @pl.loop(0, ucnt * UNITW)
        def _(j):
            pltpu.make_async_copy(ones_v, accum.at[idx_v.at[j]], sem).wait()

        plsc.subcore_barrier()
        pltpu.sync_copy(accum.at[pl.ds(sid * per_tile, per_tile)],
                        out_hbm.at[cid, pl.ds(sid * per_tile, per_tile)])

    return deg_kernel(ei3)


def _sc_aggregate(table, ei3, nwin, npad):
    per_tile = npad // NS
    split = _split(nwin)
    max_rows = (max(split[1], split[3]) + 1) * UNITW

    @functools.partial(
        pl.kernel,
        out_type=jax.ShapeDtypeStruct((NC, HIDDEN, npad), jnp.float32),
        mesh=_MESH,
        compiler_params=pltpu.CompilerParams(use_tc_tiling_on_sc=False,
                                             needs_layout_passes=False),
        scratch_types=[
            pltpu.VMEM_SHARED((npad, HIDDEN), jnp.float32),
            pltpu.VMEM((max_rows, WIN), jnp.int32),
            pltpu.VMEM((max_rows, WIN), jnp.int32),
            pltpu.VMEM((NSETS * SETW, WIN, HIDDEN), jnp.float32),
            pltpu.VMEM((per_tile, HIDDEN), jnp.float32),
            pltpu.VMEM((HIDDEN * per_tile,), jnp.float32),
            pltpu.SemaphoreType.DMA,
            pltpu.SemaphoreType.DMA,
            pltpu.SemaphoreType.DMA,
            pltpu.SemaphoreType.DMA,
            pltpu.SemaphoreType.DMA,
            pltpu.SemaphoreType.DMA,
        ],
    )
    def agg_kernel(tab_hbm, ei_hbm, out_hbm, accum, src_v, dst_v, bufs, zb,
                   tbuf, sg0, sg1, sg2, ss0, ss1, ss2):
        cid = lax.axis_index("c")
        sid = lax.axis_index("s")
        ubase, ucnt = _my_units(cid, sid, split)
        sg = (sg0, sg1, sg2)
        ss = (ss0, ss1, ss2)

        @pl.loop(0, per_tile)
        def _(i):
            zb[i, :] = jnp.zeros((HIDDEN,), jnp.float32)

        pltpu.sync_copy(zb, accum.at[pl.ds(sid * per_tile, per_tile)])
        _copy_my_windows(ei_hbm, 0, src_v, cid, sid, ubase, split)
        _copy_my_windows(ei_hbm, 1, dst_v, cid, sid, ubase, split)
        plsc.subcore_barrier()

        def fire_g(s, body):
            for b in range(SETW):
                pltpu.async_copy(tab_hbm.at[src_v.at[body * SETW + b]],
                                 bufs.at[s * SETW + b], sg[s])

        def drain_g(s):
            for b in range(SETW):
                pltpu.make_async_copy(tab_hbm.at[src_v.at[0]],
                                      bufs.at[s * SETW + b], sg[s]).wait()

        def fire_s(s, body):
            for b in range(SETW):
                pltpu.async_copy(bufs.at[s * SETW + b],
                                 accum.at[dst_v.at[body * SETW + b]],
                                 ss[s], add=True)

        def drain_s(s):
            for b in range(SETW):
                pltpu.make_async_copy(bufs.at[s * SETW + b],
                                      accum.at[dst_v.at[0]], ss[s]).wait()

        nb = ucnt * NSETS
        fire_g(0, 0)
        fire_g(1, 1)

        @pl.loop(0, ucnt)
        def _(p):
            b0 = p * NSETS

            drain_g(0)
            fire_s(0, b0)

            @pl.when(p > 0)
            def _():
                drain_s(2)

            fire_g(2, b0 + 2)
            drain_g(1)
            fire_s(1, b0 + 1)
            drain_s(0)

            @pl.when(b0 + 3 < nb)
            def _():
                fire_g(0, b0 + 3)

            drain_g(2)
            fire_s(2, b0 + 2)
            drain_s(1)

            @pl.when(b0 + 4 < nb)
            def _():
                fire_g(1, b0 + 4)

        drain_s(2)
        plsc.subcore_barrier()

        pltpu.sync_copy(accum.at[pl.ds(sid * per_tile, per_tile)], zb)
        col0 = lax.iota(jnp.int32, 16) * per_tile

        @pl.loop(0, per_tile)
        def _(i):
            plsc.store_scatter(tbuf, (col0 + i,), zb[i, :])

        for h in range(HIDDEN):
            pltpu.sync_copy(tbuf.at[pl.ds(h * per_tile, per_tile)],
                            out_hbm.at[cid, h, pl.ds(sid * per_tile, per_tile)])

    return agg_kernel(table, ei3)


def _tc_matmul(xT, W1):
    npad = xT.shape[1]

    def body(w_ref, x_ref, o_ref):
        o_ref[...] = lax.dot_general(
            w_ref[...], x_ref[...], (((1,), (0,)), ((), ())),
            preferred_element_type=jnp.float32)

    return pl.pallas_call(
        body,
        out_shape=jax.ShapeDtypeStruct((HIDDEN, npad), jnp.float32),
    )(W1, xT)


def _tc_scale(degh, h1T):
    npad = h1T.shape[1]

    def body(d_ref, h_ref, g_ref, dis_ref):
        deg = d_ref[0:1, :] + d_ref[1:2, :] + 1.0
        dis = lax.rsqrt(deg)
        dis_ref[...] = dis
        g_ref[...] = dis * h_ref[...]

    return pl.pallas_call(
        body,
        out_shape=(jax.ShapeDtypeStruct((HIDDEN, npad), jnp.float32),
                   jax.ShapeDtypeStruct((1, npad), jnp.float32)),
    )(degh, h1T)


def _tc_mid(PT, g1T, dis, b1col):
    npad = g1T.shape[1]

    def body(p_ref, g_ref, dis_ref, b_ref, o_ref):
        agg = p_ref[0] + p_ref[1] + g_ref[...]
        pre = dis_ref[...] * agg + b_ref[...]
        z = jnp.where(pre >= 0, pre, 0.01 * pre)
        o_ref[...] = dis_ref[...] * z

    return pl.pallas_call(
        body,
        out_shape=jax.ShapeDtypeStruct((HIDDEN, npad), jnp.float32),
    )(PT, g1T, dis, b1col)


def _tc_final(QT, g2T, dis, W2, b2col):
    npad = g2T.shape[1]

    def body(q_ref, g_ref, dis_ref, w_ref, b_ref, o_ref):
        agg = dis_ref[...] * (q_ref[0] + q_ref[1] + g_ref[...])
        logits = lax.dot_general(
            w_ref[...], agg, (((1,), (0,)), ((), ())),
            preferred_element_type=jnp.float32) + b_ref[...]
        m = jnp.max(logits, axis=0, keepdims=True)
        s = jnp.sum(jnp.exp(logits - m), axis=0, keepdims=True)
        o_ref[...] = logits - (m + jnp.log(s))

    return pl.pallas_call(
        body,
        out_shape=jax.ShapeDtypeStruct((N_OUT, npad), jnp.float32),
    )(QT, g2T, dis, W2, b2col)


def kernel(x, edge_index, W1, b1, W2, b2):
    n = x.shape[0]
    e = edge_index.shape[1]
    npad = -(-n // (NS * 16)) * (NS * 16)
    nwin = -(-e // (WIN * UNITW)) * UNITW
    ep = nwin * WIN

    ei = edge_index.astype(jnp.int32)
    ei3 = jnp.pad(ei, ((0, 0), (0, ep - e)), constant_values=n).reshape(
        2, nwin, WIN)
    xT = jnp.pad(x.T, ((0, 0), (0, npad - n)))

    degh = _sc_degree(ei3, nwin, npad)
    h1T = _tc_matmul(xT, W1)
    g1T, dis = _tc_scale(degh, h1T)
    P = _sc_aggregate(g1T.T, ei3, nwin, npad)
    g2T = _tc_mid(P, g1T, dis, b1.reshape(HIDDEN, 1))
    Q = _sc_aggregate(g2T.T, ei3, nwin, npad)
    outT = _tc_final(Q, g2T, dis, W2, b2.reshape(N_OUT, 1))
    return outT.T[:n]
